# baseline (speedup 1.0000x reference)
import functools

import jax
import jax.numpy as jnp
from jax import lax
from jax.experimental import pallas as pl
from jax.experimental.pallas import tpu as pltpu
from jax.experimental.pallas import tpu_sc as plsc

N = 10000
NP = 10240
E = 160000
EP = 163840
D = 32
NG = 256
STEPS = 6

NW = 32
CH = 128
ECHUNKS = EP // (NW * CH)
EBLK = 1024
KA = 144
RPT = NP // 16

_SC_MESH = dict(core_axis_name="c", subcore_axis_name="s")


def _leaky(x):
    return jnp.where(x >= 0, x, 0.01 * x)


def _make_gather(nchunks):
    rows = NW * nchunks * CH
    nbuf = 4 if nchunks % 4 == 0 else 2

    @functools.partial(
        pl.kernel,
        out_type=jax.ShapeDtypeStruct((rows, 128), jnp.float32),
        mesh=plsc.VectorSubcoreMesh(**_SC_MESH),
        scratch_types=[
            pltpu.VMEM((nchunks, CH), jnp.int32),
            [pltpu.VMEM((CH, 128), jnp.float32)] * nbuf,
            [pltpu.SemaphoreType.DMA] * nbuf,
            [pltpu.SemaphoreType.DMA] * nbuf,
        ],
    )
    def gather_k(table_hbm, idx_hbm, out_hbm, idx_v, bufs, gsems, wsems):
        c = lax.axis_index("c")
        s = lax.axis_index("s")
        wid = s * 2 + c
        base = wid * nchunks * CH
        pltpu.sync_copy(idx_hbm.at[wid], idx_v)
        for b in range(nbuf - 1):
            pltpu.async_copy(table_hbm.at[idx_v.at[b]], bufs[b], gsems[b])

        def body(t, _):
            for b in range(nbuf):
                j = t * nbuf + b
                jn = j + nbuf - 1
                bb = (b + nbuf - 1) % nbuf

                @pl.when(jn < nchunks)
                def _():
                    @pl.when(j >= 1)
                    def _():
                        pltpu.make_async_copy(
                            bufs[bb],
                            out_hbm.at[pl.ds(base + (jn - nbuf) * CH, CH)],
                            wsems[bb]).wait()
                    pltpu.async_copy(
                        table_hbm.at[idx_v.at[jn]], bufs[bb], gsems[bb])
                pltpu.make_async_copy(
                    table_hbm.at[idx_v.at[j]], bufs[b], gsems[b]).wait()
                pltpu.async_copy(
                    bufs[b], out_hbm.at[pl.ds(base + j * CH, CH)], wsems[b])
            return ()

        lax.fori_loop(0, nchunks // nbuf, body, (), unroll=False)
        for b in range(nbuf):
            j_end = nchunks - nbuf + b
            if j_end >= 0:
                pltpu.make_async_copy(
                    bufs[b], out_hbm.at[pl.ds(base + j_end * CH, CH)],
                    wsems[b]).wait()

    return gather_k


_gather_main = _make_gather(ECHUNKS)
_gather_epi = _make_gather(2)


@functools.partial(
    pl.kernel,
    out_type=jax.ShapeDtypeStruct((2, NP, 128), jnp.float32),
    mesh=plsc.VectorSubcoreMesh(**_SC_MESH),
    scratch_types=[
        pltpu.VMEM((ECHUNKS, CH), jnp.int32),
        [pltpu.VMEM((CH, 128), jnp.float32)] * 2,
        pltpu.VMEM_SHARED((NP, 128), jnp.float32),
        [pltpu.SemaphoreType.DMA] * 2,
        [pltpu.SemaphoreType.DMA] * 2,
    ],
)
def _scatter_k(msg_hbm, idx_hbm, zrow_hbm, out_hbm,
               idx_v, bufs, acc, lsems, ssems):
    nbuf = 2
    c = lax.axis_index("c")
    s = lax.axis_index("s")
    wid = s * 2 + c
    pltpu.sync_copy(zrow_hbm.at[pl.ds(s * RPT, RPT)],
                    acc.at[pl.ds(s * RPT, RPT)])
    pltpu.sync_copy(idx_hbm.at[wid], idx_v)
    plsc.subcore_barrier()
    base = wid * ECHUNKS * CH
    for b in range(nbuf - 1):
        pltpu.async_copy(msg_hbm.at[pl.ds(base + b * CH, CH)], bufs[b],
                         lsems[b])

    def body(t, _):
        for b in range(nbuf):
            j = t * nbuf + b
            jn = j + nbuf - 1
            bb = (b + nbuf - 1) % nbuf

            @pl.when(jn < ECHUNKS)
            def _():
                @pl.when(j >= 1)
                def _():
                    pltpu.make_async_copy(
                        bufs[bb], acc.at[idx_v.at[jn - nbuf]], ssems[bb]).wait()
                pltpu.async_copy(msg_hbm.at[pl.ds(base + jn * CH, CH)],
                                 bufs[bb], lsems[bb])
            pltpu.make_async_copy(msg_hbm.at[pl.ds(base + j * CH, CH)],
                                  bufs[b], lsems[b]).wait()
            pltpu.async_copy(bufs[b], acc.at[idx_v.at[j]], ssems[b], add=True)
        return ()

    lax.fori_loop(0, ECHUNKS // nbuf, body, (), unroll=False)
    for b in range(nbuf):
        j_end = ECHUNKS - nbuf + b
        pltpu.make_async_copy(bufs[b], acc.at[idx_v.at[j_end]],
                              ssems[b]).wait()
    plsc.subcore_barrier()
    pltpu.sync_copy(acc.at[pl.ds(s * RPT, RPT)],
                    out_hbm.at[c, pl.ds(s * RPT, RPT)])


def _edge_body(h_ref, xj_ref, w_ref, o_ref):
    h = h_ref[...]
    xj = xj_ref[:, 0:D]
    w = w_ref[...]
    ewT = lax.dot_general(w, h, (((1,), (0,)), ((), ())),
                          preferred_element_type=jnp.float32)
    ident = (lax.broadcasted_iota(jnp.int32, (D, D), 0)
             == lax.broadcasted_iota(jnp.int32, (D, D), 1)).astype(jnp.float32)
    xjT = lax.dot_general(ident, xj, (((1,), (1,)), ((), ())),
                          preferred_element_type=jnp.float32)
    acc = jnp.zeros((D, EBLK), jnp.float32)
    for i in range(D):
        xi = jnp.broadcast_to(xjT[i:i + 1, :], (D, EBLK))
        acc = acc + xi * ewT[i * D:(i + 1) * D, :]
    msg = lax.dot_general(acc, ident, (((0,), (0,)), ((), ())),
                          preferred_element_type=jnp.float32)
    o_ref[...] = jnp.concatenate(
        [msg, jnp.zeros((EBLK, 128 - D), jnp.float32)], axis=1)


def _edge_matvec(h128aT, xj, We2aug):
    grid = EP // EBLK
    return pl.pallas_call(
        _edge_body,
        grid=(grid,),
        in_specs=[
            pl.BlockSpec((KA, EBLK), lambda i: (0, i)),
            pl.BlockSpec((EBLK, 128), lambda i: (i, 0)),
            pl.BlockSpec((D * D, KA), lambda i: (0, 0)),
        ],
        out_specs=pl.BlockSpec((EBLK, 128), lambda i: (i, 0)),
        out_shape=jax.ShapeDtypeStruct((EP, 128), jnp.float32),
    )(h128aT, xj, We2aug)


def kernel(x, edge_index, edge_attr, batch, stem_atmidx, jbond_atmidx, W0, b0, We1, be1, We2, be2, Wroot, cbias, Wih, Whh, bih, bhh, W1, b1, lWih, lWhh, lbih, lbhh, W3, b3, Ws1, bs1, Ws2, bs2, Wj1, bj1, Wj2, bj2):
    src = jnp.pad(edge_index[0], (0, EP - E))
    dst = jnp.pad(edge_index[1], (0, EP - E),
                  constant_values=NP - 1)
    src_l = src.reshape(NW, ECHUNKS, CH)
    dst_l = dst.reshape(NW, ECHUNKS, CH)

    xP = jnp.pad(x, ((0, NP - N), (0, 0)))
    out = _leaky(xP @ W0.T + b0)
    h = out

    eaP = jnp.pad(edge_attr, ((0, EP - E), (0, 0)))
    h128 = _leaky(eaP @ We1.T + be1)
    h128aT = jnp.concatenate([
        h128.T.astype(jnp.bfloat16),
        jnp.ones((1, EP), jnp.bfloat16),
        jnp.zeros((KA - 129, EP), jnp.bfloat16),
    ], axis=0)
    We2aug = jnp.concatenate([
        We2.astype(jnp.bfloat16),
        be2[:, None].astype(jnp.bfloat16),
        jnp.zeros((D * D, KA - 129), jnp.bfloat16),
    ], axis=1)

    zrow = jnp.zeros((NP, 128), jnp.float32)
    onesE = jnp.ones((EP, 128), jnp.float32)
    deg2 = _scatter_k(onesE, dst_l, zrow)
    deg = jnp.maximum(deg2[0, :, 0] + deg2[1, :, 0], 1.0)[:, None]

    for _ in range(STEPS):
        out128 = jnp.pad(out, ((0, 0), (0, 128 - D)))
        xj = _gather_main(out128, src_l)
        msg = _edge_matvec(h128aT, xj, We2aug)
        agg2 = _scatter_k(msg, dst_l, zrow)
        agg = agg2[0, :, :D] + agg2[1, :, :D]
        agg = agg / deg
        m = _leaky(agg + out @ Wroot.T + cbias)
        gi = m @ Wih.T + bih
        gh = h @ Whh.T + bhh
        ir, iz, inn = jnp.split(gi, 3, axis=-1)
        hr, hz, hn = jnp.split(gh, 3, axis=-1)
        r = jax.nn.sigmoid(ir + hr)
        z = jax.nn.sigmoid(iz + hz)
        n = jnp.tanh(inn + r * hn)
        out = (1.0 - z) * n + z * h
        h = out

    idx_epi = jnp.concatenate([
        stem_atmidx, jnp.zeros((48,), jnp.int32),
        jbond_atmidx[:, 0], jnp.zeros((4,), jnp.int32),
        jbond_atmidx[:, 1], jnp.zeros((4,), jnp.int32),
        jnp.zeros((8192 - 5056,), jnp.int32),
    ])
    out128 = jnp.pad(out, ((0, 0), (0, 128 - D)))
    g_epi = _gather_epi(out128, idx_epi.reshape(NW, 2, CH))[:, :D]
    pa_s = _leaky(g_epi[:2048] @ W1.T + b1)
    stem_preds = (_leaky(pa_s @ Ws1.T + bs1) @ Ws2.T + bs2)[:2000]
    pa_j = _leaky(g_epi[2048:5056] @ W1.T + b1)
    vj = (_leaky(pa_j @ Wj1.T + bj1) @ Wj2.T + bj2)
    jbond_preds = (0.5 * (vj[:1504] + vj[1504:]))[:1500, 0]

    outN = out[:N]
    qstar = jnp.zeros((NG, 2 * D), jnp.float32)
    hh = jnp.zeros((NG, D), jnp.float32)
    cc = jnp.zeros((NG, D), jnp.float32)
    for _ in range(3):
        g = qstar @ lWih.T + lbih + hh @ lWhh.T + lbhh
        i, f, gg, o = jnp.split(g, 4, axis=-1)
        i = jax.nn.sigmoid(i)
        f = jax.nn.sigmoid(f)
        o = jax.nn.sigmoid(o)
        gg = jnp.tanh(gg)
        cc = f * cc + i * gg
        hh = o * jnp.tanh(cc)
        q = hh
        e = jnp.sum(outN * q[batch], axis=-1)
        emax = jax.ops.segment_max(e, batch, num_segments=NG)
        ee = jnp.exp(e - emax[batch])
        denom = jax.ops.segment_sum(ee, batch, num_segments=NG)
        a = ee / (denom[batch] + 1e-16)
        r = jax.ops.segment_sum(a[:, None] * outN, batch, num_segments=NG)
        qstar = jnp.concatenate([q, r], axis=-1)
    sout = qstar @ W3.T + b3
    return sout, stem_preds, jbond_preds


# speedup vs baseline: 1.2023x; 1.2023x over previous
import functools

import jax
import jax.numpy as jnp
from jax import lax
from jax.experimental import pallas as pl
from jax.experimental.pallas import tpu as pltpu
from jax.experimental.pallas import tpu_sc as plsc

N = 10000
NP = 10240
E = 160000
EP = 163840
D = 32
NG = 256
STEPS = 6

NW = 32
CH = 128
ECHUNKS = EP // (NW * CH)
EBLK = 1024
KA = 144
RPT = NP // 16

_SC_MESH = dict(core_axis_name="c", subcore_axis_name="s")


def _leaky(x):
    return jnp.where(x >= 0, x, 0.01 * x)


def _make_gather(nchunks):
    rows = NW * nchunks * CH
    nbuf = 4 if nchunks % 4 == 0 else 2

    @functools.partial(
        pl.kernel,
        out_type=jax.ShapeDtypeStruct((rows, 128), jnp.float32),
        mesh=plsc.VectorSubcoreMesh(**_SC_MESH),
        scratch_types=[
            pltpu.VMEM((nchunks, CH), jnp.int32),
            [pltpu.VMEM((CH, 128), jnp.float32)] * nbuf,
            [pltpu.SemaphoreType.DMA] * nbuf,
            [pltpu.SemaphoreType.DMA] * nbuf,
        ],
    )
    def gather_k(table_hbm, idx_hbm, out_hbm, idx_v, bufs, gsems, wsems):
        c = lax.axis_index("c")
        s = lax.axis_index("s")
        wid = s * 2 + c
        base = wid * nchunks * CH
        pltpu.sync_copy(idx_hbm.at[wid], idx_v)
        for b in range(nbuf - 1):
            pltpu.async_copy(table_hbm.at[idx_v.at[b]], bufs[b], gsems[b])

        def body(t, _):
            for b in range(nbuf):
                j = t * nbuf + b
                jn = j + nbuf - 1
                bb = (b + nbuf - 1) % nbuf

                @pl.when(jn < nchunks)
                def _():
                    @pl.when(j >= 1)
                    def _():
                        pltpu.make_async_copy(
                            bufs[bb],
                            out_hbm.at[pl.ds(base + (jn - nbuf) * CH, CH)],
                            wsems[bb]).wait()
                    pltpu.async_copy(
                        table_hbm.at[idx_v.at[jn]], bufs[bb], gsems[bb])
                pltpu.make_async_copy(
                    table_hbm.at[idx_v.at[j]], bufs[b], gsems[b]).wait()
                pltpu.async_copy(
                    bufs[b], out_hbm.at[pl.ds(base + j * CH, CH)], wsems[b])
            return ()

        lax.fori_loop(0, nchunks // nbuf, body, (), unroll=False)
        for b in range(nbuf):
            j_end = nchunks - nbuf + b
            if j_end >= 0:
                pltpu.make_async_copy(
                    bufs[b], out_hbm.at[pl.ds(base + j_end * CH, CH)],
                    wsems[b]).wait()

    return gather_k


_gather_main = _make_gather(ECHUNKS)
_gather_epi = _make_gather(2)


@functools.partial(
    pl.kernel,
    out_type=jax.ShapeDtypeStruct((2, NP, 128), jnp.float32),
    mesh=plsc.VectorSubcoreMesh(**_SC_MESH),
    scratch_types=[
        pltpu.VMEM((ECHUNKS, CH), jnp.int32),
        [pltpu.VMEM((CH, 128), jnp.float32)] * 2,
        pltpu.VMEM_SHARED((NP, 128), jnp.float32),
        [pltpu.SemaphoreType.DMA] * 2,
        [pltpu.SemaphoreType.DMA] * 2,
    ],
)
def _scatter_k(msg_hbm, idx_hbm, zrow_hbm, out_hbm,
               idx_v, bufs, acc, lsems, ssems):
    nbuf = 2
    c = lax.axis_index("c")
    s = lax.axis_index("s")
    wid = s * 2 + c
    pltpu.sync_copy(zrow_hbm.at[pl.ds(s * RPT, RPT)],
                    acc.at[pl.ds(s * RPT, RPT)])
    pltpu.sync_copy(idx_hbm.at[wid], idx_v)
    plsc.subcore_barrier()
    base = wid * ECHUNKS * CH
    for b in range(nbuf - 1):
        pltpu.async_copy(msg_hbm.at[pl.ds(base + b * CH, CH)], bufs[b],
                         lsems[b])

    def body(t, _):
        for b in range(nbuf):
            j = t * nbuf + b
            jn = j + nbuf - 1
            bb = (b + nbuf - 1) % nbuf

            @pl.when(jn < ECHUNKS)
            def _():
                @pl.when(j >= 1)
                def _():
                    pltpu.make_async_copy(
                        bufs[bb], acc.at[idx_v.at[jn - nbuf]], ssems[bb]).wait()
                pltpu.async_copy(msg_hbm.at[pl.ds(base + jn * CH, CH)],
                                 bufs[bb], lsems[bb])
            pltpu.make_async_copy(msg_hbm.at[pl.ds(base + j * CH, CH)],
                                  bufs[b], lsems[b]).wait()
            pltpu.async_copy(bufs[b], acc.at[idx_v.at[j]], ssems[b], add=True)
        return ()

    lax.fori_loop(0, ECHUNKS // nbuf, body, (), unroll=False)
    for b in range(nbuf):
        j_end = ECHUNKS - nbuf + b
        pltpu.make_async_copy(bufs[b], acc.at[idx_v.at[j_end]],
                              ssems[b]).wait()
    plsc.subcore_barrier()
    pltpu.sync_copy(acc.at[pl.ds(s * RPT, RPT)],
                    out_hbm.at[c, pl.ds(s * RPT, RPT)])


def _edge_body(h_ref, xj_ref, w_ref, o_ref):
    h = h_ref[...]
    xj = xj_ref[:, 0:D]
    w = w_ref[...]
    ewT = lax.dot_general(w, h, (((1,), (0,)), ((), ())),
                          preferred_element_type=jnp.float32)
    ident = (lax.broadcasted_iota(jnp.int32, (D, D), 0)
             == lax.broadcasted_iota(jnp.int32, (D, D), 1)).astype(jnp.float32)
    xjT = lax.dot_general(ident, xj, (((1,), (1,)), ((), ())),
                          preferred_element_type=jnp.float32)
    acc = jnp.zeros((D, EBLK), jnp.float32)
    for i in range(D):
        xi = jnp.broadcast_to(xjT[i:i + 1, :], (D, EBLK))
        acc = acc + xi * ewT[i * D:(i + 1) * D, :]
    msg = lax.dot_general(acc, ident, (((0,), (0,)), ((), ())),
                          preferred_element_type=jnp.float32)
    o_ref[...] = jnp.concatenate(
        [msg, jnp.ones((EBLK, 1), jnp.float32),
         jnp.zeros((EBLK, 127 - D), jnp.float32)], axis=1)


def _edge_matvec(h128aT, xj, We2aug):
    grid = EP // EBLK
    return pl.pallas_call(
        _edge_body,
        grid=(grid,),
        in_specs=[
            pl.BlockSpec((KA, EBLK), lambda i: (0, i)),
            pl.BlockSpec((EBLK, 128), lambda i: (i, 0)),
            pl.BlockSpec((D * D, KA), lambda i: (0, 0)),
        ],
        out_specs=pl.BlockSpec((EBLK, 128), lambda i: (i, 0)),
        out_shape=jax.ShapeDtypeStruct((EP, 128), jnp.float32),
    )(h128aT, xj, We2aug)


RB = 2048


def _out0_body(x_ref, w_ref, b_ref, oc_ref, op_ref):
    o = _leaky(lax.dot_general(x_ref[...], w_ref[...], (((1,), (0,)), ((), ())),
                               preferred_element_type=jnp.float32) + b_ref[...])
    oc_ref[...] = o
    op_ref[...] = jnp.concatenate(
        [o, jnp.zeros((RB, 128 - D), jnp.float32)], axis=1)


def _out0(xP16, W0T16, b0r):
    return pl.pallas_call(
        _out0_body,
        grid=(NP // RB,),
        in_specs=[
            pl.BlockSpec((RB, 16), lambda i: (i, 0)),
            pl.BlockSpec((16, D), lambda i: (0, 0)),
            pl.BlockSpec((1, D), lambda i: (0, 0)),
        ],
        out_specs=[
            pl.BlockSpec((RB, D), lambda i: (i, 0)),
            pl.BlockSpec((RB, 128), lambda i: (i, 0)),
        ],
        out_shape=[
            jax.ShapeDtypeStruct((NP, D), jnp.float32),
            jax.ShapeDtypeStruct((NP, 128), jnp.float32),
        ],
    )(xP16, W0T16, b0r)


EB2 = 4096


def _h128_body(w_ref, ea_ref, o_ref):
    h = _leaky(lax.dot_general(w_ref[...], ea_ref[...], (((1,), (0,)), ((), ())),
                               preferred_element_type=jnp.float32))
    o_ref[...] = jnp.concatenate([
        h.astype(jnp.bfloat16),
        jnp.ones((1, EB2), jnp.bfloat16),
        jnp.zeros((KA - 129, EB2), jnp.bfloat16),
    ], axis=0)


def _h128T(We1aug, eaT8):
    return pl.pallas_call(
        _h128_body,
        grid=(EP // EB2,),
        in_specs=[
            pl.BlockSpec((128, 8), lambda i: (0, 0)),
            pl.BlockSpec((8, EB2), lambda i: (0, i)),
        ],
        out_specs=pl.BlockSpec((KA, EB2), lambda i: (0, i)),
        out_shape=jax.ShapeDtypeStruct((KA, EP), jnp.bfloat16),
    )(We1aug, eaT8)


def _gru_body(a_ref, dinv_ref, o_ref, wr_ref, cb_ref,
              wir_ref, whr_ref, br_ref, wiz_ref, whz_ref, bz_ref,
              win_ref, whn_ref, bn_ref, bhn_ref, oc_ref, op_ref):
    def mm(a, b):
        return lax.dot_general(a, b, (((1,), (0,)), ((), ())),
                               preferred_element_type=jnp.float32)
    agg = (a_ref[0, :, 0:D] + a_ref[1, :, 0:D]) * dinv_ref[:, 0:D]
    o = o_ref[...]
    m = _leaky(agg + mm(o, wr_ref[...]) + cb_ref[...])
    r = jax.nn.sigmoid(mm(m, wir_ref[...]) + mm(o, whr_ref[...]) + br_ref[...])
    z = jax.nn.sigmoid(mm(m, wiz_ref[...]) + mm(o, whz_ref[...]) + bz_ref[...])
    n = jnp.tanh(mm(m, win_ref[...]) + bn_ref[...]
                 + r * (mm(o, whn_ref[...]) + bhn_ref[...]))
    newo = (1.0 - z) * n + z * o
    oc_ref[...] = newo
    op_ref[...] = jnp.concatenate(
        [newo, jnp.zeros((RB, 128 - D), jnp.float32)], axis=1)


def _gru(agg2, dinv, out_c, wmats):
    specs = [
        pl.BlockSpec((2, RB, 128), lambda i: (0, i, 0)),
        pl.BlockSpec((RB, 128), lambda i: (i, 0)),
        pl.BlockSpec((RB, D), lambda i: (i, 0)),
    ]
    for wm in wmats:
        if wm.shape == (D, D):
            specs.append(pl.BlockSpec((D, D), lambda i: (0, 0)))
        else:
            specs.append(pl.BlockSpec((1, D), lambda i: (0, 0)))
    return pl.pallas_call(
        _gru_body,
        grid=(NP // RB,),
        in_specs=specs,
        out_specs=[
            pl.BlockSpec((RB, D), lambda i: (i, 0)),
            pl.BlockSpec((RB, 128), lambda i: (i, 0)),
        ],
        out_shape=[
            jax.ShapeDtypeStruct((NP, D), jnp.float32),
            jax.ShapeDtypeStruct((NP, 128), jnp.float32),
        ],
    )(agg2, dinv, out_c, *wmats)


def _stem_body(g_ref, w1_ref, b1_ref, ws1_ref, bs1_ref, ws2_ref, bs2_ref, o_ref):
    def mm(a, b):
        return lax.dot_general(a, b, (((1,), (0,)), ((), ())),
                               preferred_element_type=jnp.float32)
    pa = _leaky(mm(g_ref[:, 0:D], w1_ref[...]) + b1_ref[...])
    o_ref[...] = mm(_leaky(mm(pa, ws1_ref[...]) + bs1_ref[...]),
                    ws2_ref[...]) + bs2_ref[...]


def _stem_head(g, W1T, b1r, Ws1T, bs1r, Ws2T, bs2r):
    return pl.pallas_call(
        _stem_body,
        in_specs=[pl.BlockSpec(g.shape, lambda: (0, 0)),
                  pl.BlockSpec((D, 256), lambda: (0, 0)),
                  pl.BlockSpec((1, 256), lambda: (0, 0)),
                  pl.BlockSpec((256, D), lambda: (0, 0)),
                  pl.BlockSpec((1, D), lambda: (0, 0)),
                  pl.BlockSpec((D, 105), lambda: (0, 0)),
                  pl.BlockSpec((1, 105), lambda: (0, 0))],
        out_specs=pl.BlockSpec((g.shape[0], 105), lambda: (0, 0)),
        out_shape=jax.ShapeDtypeStruct((g.shape[0], 105), jnp.float32),
    )(g, W1T, b1r, Ws1T, bs1r, Ws2T, bs2r)


def _jbond_body(g_ref, w1_ref, b1_ref, wj1_ref, bj1_ref, wj2_ref, bj2_ref, o_ref):
    def mm(a, b):
        return lax.dot_general(a, b, (((1,), (0,)), ((), ())),
                               preferred_element_type=jnp.float32)
    pa = _leaky(mm(g_ref[:, 0:D], w1_ref[...]) + b1_ref[...])
    vj = mm(_leaky(mm(pa, wj1_ref[...]) + bj1_ref[...]),
            wj2_ref[...]) + bj2_ref[...]
    o_ref[...] = 0.5 * (vj[0:1504] + vj[1504:3008])


def _jbond_head(g, W1T, b1r, Wj1T, bj1r, Wj2T8, bj2r8):
    return pl.pallas_call(
        _jbond_body,
        in_specs=[pl.BlockSpec(g.shape, lambda: (0, 0)),
                  pl.BlockSpec((D, 256), lambda: (0, 0)),
                  pl.BlockSpec((1, 256), lambda: (0, 0)),
                  pl.BlockSpec((256, D), lambda: (0, 0)),
                  pl.BlockSpec((1, D), lambda: (0, 0)),
                  pl.BlockSpec((D, 8), lambda: (0, 0)),
                  pl.BlockSpec((1, 8), lambda: (0, 0))],
        out_specs=pl.BlockSpec((1504, 8), lambda: (0, 0)),
        out_shape=jax.ShapeDtypeStruct((1504, 8), jnp.float32),
    )(g, W1T, b1r, Wj1T, bj1r, Wj2T8, bj2r8)


def _set2set_body(oc_ref, b_ref, lw_refs, w3_ref, b3_ref, o_ref, s_ref):
    (wiI, whI, bI, wiF, whF, bF, wiG, whG, bG, wiO, whO, bO) = lw_refs

    def mm(a, b):
        return lax.dot_general(a, b, (((1,), (0,)), ((), ())),
                               preferred_element_type=jnp.float32)
    out = oc_ref[...]
    bcol = b_ref[:, 0:1]
    s_ref[...] = (jnp.broadcast_to(bcol, (NP, NG))
                  == lax.broadcasted_iota(jnp.int32, (NP, NG), 1)
                  ).astype(jnp.float32)
    qs = jnp.zeros((NG, 2 * D), jnp.float32)
    hh = jnp.zeros((NG, D), jnp.float32)
    cc = jnp.zeros((NG, D), jnp.float32)
    for _ in range(3):
        i = jax.nn.sigmoid(mm(qs, wiI[...]) + mm(hh, whI[...]) + bI[...])
        f = jax.nn.sigmoid(mm(qs, wiF[...]) + mm(hh, whF[...]) + bF[...])
        gg = jnp.tanh(mm(qs, wiG[...]) + mm(hh, whG[...]) + bG[...])
        o = jax.nn.sigmoid(mm(qs, wiO[...]) + mm(hh, whO[...]) + bO[...])
        cc = f * cc + i * gg
        hh = o * jnp.tanh(cc)
        q = hh
        S = s_ref[...]
        qb = mm(S, q)
        e = jnp.sum(out * qb, axis=1, keepdims=True)
        eb = jnp.broadcast_to(e, (NP, NG))
        emax_row = jnp.max(jnp.where(S > 0, eb, -1e30), axis=0, keepdims=True)
        emax_b = jnp.max(jnp.where(S > 0, jnp.broadcast_to(emax_row, (NP, NG)),
                                   -1e30), axis=1, keepdims=True)
        ee = jnp.exp(jnp.minimum(e - emax_b, 0.0))
        den_row = jnp.sum(S * ee, axis=0, keepdims=True)
        den_b = jnp.sum(S * den_row, axis=1, keepdims=True)
        a = ee / (den_b + 1e-16)
        r = lax.dot_general(S, a * out, (((0,), (0,)), ((), ())),
                            preferred_element_type=jnp.float32)
        qs = jnp.concatenate([q, r], axis=1)
    o_ref[...] = mm(qs, w3_ref[...]) + b3_ref[...]


def _set2set(out_c, batch8, lws, W3T8, b3r8):
    in_specs = [pl.BlockSpec((NP, D), lambda: (0, 0)),
                pl.BlockSpec((NP, 8), lambda: (0, 0))]
    for wm in lws:
        in_specs.append(pl.BlockSpec(wm.shape, lambda: (0, 0)))
    in_specs.append(pl.BlockSpec((2 * D, 8), lambda: (0, 0)))
    in_specs.append(pl.BlockSpec((1, 8), lambda: (0, 0)))

    def body(oc_ref, b_ref, *rest):
        lw_refs = rest[:12]
        w3_ref, b3_ref, o_ref, s_ref = rest[12:]
        _set2set_body(oc_ref, b_ref, lw_refs, w3_ref, b3_ref, o_ref, s_ref)

    return pl.pallas_call(
        body,
        in_specs=in_specs,
        out_specs=pl.BlockSpec((NG, 8), lambda: (0, 0)),
        out_shape=jax.ShapeDtypeStruct((NG, 8), jnp.float32),
        scratch_shapes=[pltpu.VMEM((NP, NG), jnp.float32)],
    )(out_c, batch8, *lws, W3T8, b3r8)


def kernel(x, edge_index, edge_attr, batch, stem_atmidx, jbond_atmidx, W0, b0, We1, be1, We2, be2, Wroot, cbias, Wih, Whh, bih, bhh, W1, b1, lWih, lWhh, lbih, lbhh, W3, b3, Ws1, bs1, Ws2, bs2, Wj1, bj1, Wj2, bj2):
    src = jnp.pad(edge_index[0], (0, EP - E))
    dst = jnp.pad(edge_index[1], (0, EP - E),
                  constant_values=NP - 1)
    src_l = src.reshape(NW, ECHUNKS, CH)
    dst_l = dst.reshape(NW, ECHUNKS, CH)

    xP16 = jnp.pad(x, ((0, NP - N), (0, 16 - x.shape[1])))
    out_c, out_p = _out0(xP16, jnp.pad(W0.T, ((0, 2), (0, 0))), b0[None, :])

    eaT8 = jnp.concatenate([
        jnp.pad(edge_attr, ((0, EP - E), (0, 0))).T,
        jnp.ones((1, EP), jnp.float32),
        jnp.zeros((3, EP), jnp.float32),
    ], axis=0)
    We1aug = jnp.concatenate(
        [We1, be1[:, None], jnp.zeros((128, 3), jnp.float32)], axis=1)
    h128aT = _h128T(We1aug, eaT8)
    We2aug = jnp.concatenate([
        We2.astype(jnp.bfloat16),
        be2[:, None].astype(jnp.bfloat16),
        jnp.zeros((D * D, KA - 129), jnp.bfloat16),
    ], axis=1)

    zrow = jnp.zeros((NP, 128), jnp.float32)
    Wih_r, Wih_z, Wih_n = jnp.split(Wih, 3, axis=0)
    Whh_r, Whh_z, Whh_n = jnp.split(Whh, 3, axis=0)
    bih_r, bih_z, bih_n = jnp.split(bih, 3)
    bhh_r, bhh_z, bhh_n = jnp.split(bhh, 3)
    gru_w = [Wroot.T, cbias[None, :],
             Wih_r.T, Whh_r.T, (bih_r + bhh_r)[None, :],
             Wih_z.T, Whh_z.T, (bih_z + bhh_z)[None, :],
             Wih_n.T, Whh_n.T, bih_n[None, :], bhh_n[None, :]]
    dinv = None
    for step in range(STEPS):
        xj = _gather_main(out_p, src_l)
        msg = _edge_matvec(h128aT, xj, We2aug)
        agg2 = _scatter_k(msg, dst_l, zrow)
        if dinv is None:
            cnt = jnp.maximum(agg2[0, :, D] + agg2[1, :, D], 1.0)
            dinv = jnp.broadcast_to((1.0 / cnt)[:, None], (NP, 128))
        out_c, out_p = _gru(agg2, dinv, out_c, gru_w)

    idx_epi = jnp.concatenate([
        stem_atmidx, jnp.zeros((48,), jnp.int32),
        jbond_atmidx[:, 0], jnp.zeros((4,), jnp.int32),
        jbond_atmidx[:, 1], jnp.zeros((4,), jnp.int32),
        jnp.zeros((8192 - 5056,), jnp.int32),
    ])
    g_epi = _gather_epi(out_p, idx_epi.reshape(NW, 2, CH))
    stem_preds = _stem_head(g_epi[:2048], W1.T, b1[None, :],
                            Ws1.T, bs1[None, :], Ws2.T, bs2[None, :])[:2000]
    jb = _jbond_head(g_epi[2048:5056], W1.T, b1[None, :],
                     Wj1.T, bj1[None, :],
                     jnp.pad(Wj2.T, ((0, 0), (0, 7))),
                     jnp.pad(bj2, (0, 7))[None, :])
    jbond_preds = jb[:1500, 0]

    lWih_i, lWih_f, lWih_g, lWih_o = jnp.split(lWih, 4, axis=0)
    lWhh_i, lWhh_f, lWhh_g, lWhh_o = jnp.split(lWhh, 4, axis=0)
    lb_i, lb_f, lb_g, lb_o = jnp.split(lbih + lbhh, 4)
    lws = [lWih_i.T, lWhh_i.T, lb_i[None, :],
           lWih_f.T, lWhh_f.T, lb_f[None, :],
           lWih_g.T, lWhh_g.T, lb_g[None, :],
           lWih_o.T, lWhh_o.T, lb_o[None, :]]
    batch8 = jnp.broadcast_to(
        jnp.pad(batch, (0, NP - N), constant_values=NG)[:, None], (NP, 8))
    sout = _set2set(out_c, batch8, lws,
                    jnp.pad(W3.T, ((0, 0), (0, 6))),
                    jnp.pad(b3, (0, 6))[None, :])[:, :2]
    return sout, stem_preds, jbond_preds
